# Initial kernel scaffold; baseline (speedup 1.0000x reference)
import jax
import jax.numpy as jnp
from jax.experimental import pallas as pl


def kernel(xs, edge_attrs, params, edges, node_batch, edge_batch):
    raise NotImplementedError("write your pallas kernel here")


# speedup vs baseline: 6.4451x; 6.4451x over previous
import functools

import jax
import jax.numpy as jnp
from jax import lax
from jax.experimental import pallas as pl
from jax.experimental.pallas import tpu as pltpu
from jax.experimental.pallas import tpu_sc as plsc

_ETYPES = [
    ('Package_Name', 'Action', 'Path'),
    ('Package_Name', 'DNS', 'DNS Host'),
    ('Package_Name', 'CMD', 'Command'),
    ('Package_Name', 'socket_ip', 'IP'),
    ('Package_Name', 'socket_port', 'Port'),
    ('Package_Name', 'socket_host', 'Hostnames'),
]
_H = 4
_C = 64
_HC = 256
_G = 2048
_TW = 272
_RT = 288
_RTP = 256
_KS = 32
_KP = 96


def _rup(x, m):
    return ((x + m - 1) // m) * m


def _mm_body(x_ref, w_ref, b_ref, o_ref):
    o_ref[...] = (
        jnp.dot(x_ref[...], w_ref[...], preferred_element_type=jnp.float32)
        + b_ref[...]
    )


def _mm(x, w, b=None):
    m, k = x.shape
    n = w.shape[1]
    if b is None:
        b = jnp.zeros((n,), jnp.float32)
    b2 = b.reshape(1, n)
    bm = min(512, _rup(m, 8))
    return pl.pallas_call(
        _mm_body,
        grid=(pl.cdiv(m, bm),),
        in_specs=[
            pl.BlockSpec((bm, k), lambda i: (i, 0)),
            pl.BlockSpec((k, n), lambda i: (0, 0)),
            pl.BlockSpec((1, n), lambda i: (0, 0)),
        ],
        out_specs=pl.BlockSpec((bm, n), lambda i: (i, 0)),
        out_shape=jax.ShapeDtypeStruct((m, n), jnp.float32),
    )(x, w, b2)


def _ln_relu_tail(x, g, b):
    mu = jnp.mean(x, -1, keepdims=True)
    var = jnp.mean((x - mu) * (x - mu), -1, keepdims=True)
    y = (x - mu) / jnp.sqrt(var + 1e-5) * g + b
    return jnp.maximum(y, 0.0)


def _lnrelu_body(x_ref, g_ref, b_ref, o_ref):
    o_ref[...] = _ln_relu_tail(x_ref[...], g_ref[...], b_ref[...])


def _ln_relu(x, g, b):
    m, n = x.shape
    bm = min(512, _rup(m, 8))
    return pl.pallas_call(
        _lnrelu_body,
        grid=(pl.cdiv(m, bm),),
        in_specs=[
            pl.BlockSpec((bm, n), lambda i: (i, 0)),
            pl.BlockSpec((1, n), lambda i: (0, 0)),
            pl.BlockSpec((1, n), lambda i: (0, 0)),
        ],
        out_specs=pl.BlockSpec((bm, n), lambda i: (i, 0)),
        out_shape=jax.ShapeDtypeStruct((m, n), jnp.float32),
    )(x, g.reshape(1, n), b.reshape(1, n))


def _gatfin_body(t_ref, bias_ref, g_ref, b_ref, o_ref):
    t = t_ref[...]
    cols = []
    for h in range(_H):
        num = t[:, h * _C:(h + 1) * _C]
        den = t[:, _HC + h:_HC + h + 1]
        cols.append(num / (den + 1e-16))
    x = jnp.concatenate(cols, axis=1) + bias_ref[...]
    o_ref[...] = _ln_relu_tail(x, g_ref[...], b_ref[...])


def _gat_finalize(table_flat, bias, g, b, n):
    bm = min(512, _rup(n, 8))
    return pl.pallas_call(
        _gatfin_body,
        grid=(pl.cdiv(n, bm),),
        in_specs=[
            pl.BlockSpec((bm, _TW), lambda i: (i, 0)),
            pl.BlockSpec((1, _HC), lambda i: (0, 0)),
            pl.BlockSpec((1, _HC), lambda i: (0, 0)),
            pl.BlockSpec((1, _HC), lambda i: (0, 0)),
        ],
        out_specs=pl.BlockSpec((bm, _HC), lambda i: (i, 0)),
        out_shape=jax.ShapeDtypeStruct((n, _HC), jnp.float32),
    )(table_flat, bias.reshape(1, _HC), g.reshape(1, _HC), b.reshape(1, _HC))


def _readout_body(*refs):
    nrefs = refs[:6]
    erefs = refs[6:12]
    wn_ref, we_ref, b_ref, o_ref = refs[12:]
    nacc = None
    for r in nrefs:
        t = r[0] + r[1] + r[2] + r[3]
        p = t[:, :_HC] / (t[:, _HC:_HC + 1] + 1e-16)
        nacc = p if nacc is None else nacc + p
    eacc = None
    for r in erefs:
        t = r[0] + r[1] + r[2] + r[3]
        p = t[:, :16] / (t[:, 16:17] + 1e-16)
        eacc = p if eacc is None else eacc + p
    nacc = nacc * (1.0 / 6.0)
    eacc = eacc * (1.0 / 6.0)
    o_ref[...] = (
        jnp.dot(nacc, wn_ref[...], preferred_element_type=jnp.float32)
        + jnp.dot(eacc, we_ref[...], preferred_element_type=jnp.float32)
        + b_ref[...]
    )


def _readout(ntabs, etabs, w, b):
    bm = 256
    wn = w[:_HC]
    we = w[_HC:]
    specs = (
        [pl.BlockSpec((4, bm, _TW), lambda i: (0, i, 0)) for _ in range(6)]
        + [pl.BlockSpec((4, bm, 32), lambda i: (0, i, 0)) for _ in range(6)]
        + [
            pl.BlockSpec((_HC, 1), lambda i: (0, 0)),
            pl.BlockSpec((16, 1), lambda i: (0, 0)),
            pl.BlockSpec((1, 1), lambda i: (0, 0)),
        ]
    )
    return pl.pallas_call(
        _readout_body,
        grid=(_G // bm,),
        in_specs=specs,
        out_specs=pl.BlockSpec((bm, 1), lambda i: (i, 0)),
        out_shape=jax.ShapeDtypeStruct((_G, 1), jnp.float32),
    )(*ntabs, *etabs, wn, we, b.reshape(1, 1))


def _iota16():
    return lax.iota(jnp.int32, 16)


def _perm(v, idx):
    dnums = lax.GatherDimensionNumbers(
        offset_dims=(), collapsed_slice_dims=(0,), start_index_map=(0,))
    return lax.gather(v, idx[:, None], dnums, (1,),
                      mode=lax.GatherScatterMode.PROMISE_IN_BOUNDS)


def _allsum(v):
    it = _iota16()
    for k in (8, 4, 2, 1):
        v = v + _perm(v, it ^ k)
    return v


def _bcast_lane(v, lane):
    return _perm(v, jnp.full((16,), lane, jnp.int32))


def _extract(v, lane):
    return lax.squeeze(lax.slice(_bcast_lane(v, lane), (0,), (1,)), (0,))


@functools.cache
def _build_edge_kernel(n_tgt, e_pad, e_real):
    nr = -(-n_tgt // _RT)
    nk = -(-nr // 32)
    mesh = plsc.VectorSubcoreMesh(core_axis_name="c", subcore_axis_name="s")

    @functools.partial(
        pl.kernel,
        out_type=jax.ShapeDtypeStruct((nr * _RT, _TW), jnp.float32),
        mesh=mesh,
        compiler_params=pltpu.CompilerParams(needs_layout_passes=False),
        scratch_types=[
            pltpu.VMEM((_KS,), jnp.int32),
            pltpu.VMEM((_KS,), jnp.int32),
            pltpu.VMEM((_KS, _HC), jnp.float32),
            pltpu.VMEM((_KS, _HC), jnp.float32),
            pltpu.VMEM((_HC,), jnp.float32),
            pltpu.VMEM((96,), jnp.int32),
            pltpu.VMEM((96,), jnp.int32),
            pltpu.VMEM((_RT, _TW), jnp.float32),
            pltpu.SemaphoreType.DMA,
        ],
    )
    def k(l_hbm, r_hbm, src_hbm, dst_hbm, att_hbm, lo_hbm, hi_hbm, out_hbm,
          idx_s, idx_d, lb, rb, attb, lob, hib, table, sem):
        wid = lax.axis_index("c") * 16 + lax.axis_index("s")
        pltpu.sync_copy(att_hbm, attb)
        pltpu.sync_copy(lo_hbm, lob)
        pltpu.sync_copy(hi_hbm, hib)
        it = _iota16()
        for k0 in range(nk):
            ri = wid + 32 * k0

            @pl.when(ri < nr)
            def _range():
                r0 = ri * _RT
                gg = (ri // 16) * 16
                ll = ri - gg
                lo = pl.multiple_of(_extract(lob[pl.ds(gg, 16)], ll), 8)
                hi = _extract(hib[pl.ds(gg, 16)], ll)

                def _zero(i, c):
                    for q in range(_TW // 16):
                        table[i, 16 * q:16 * (q + 1)] = jnp.zeros(
                            (16,), jnp.float32)
                    return c

                lax.fori_loop(0, _RT, _zero, 0)
                nch = (hi - lo + _KS - 1) // _KS

                def _chunk(j, c):
                    base = pl.multiple_of(lo + j * _KS, 8)
                    pltpu.sync_copy(src_hbm.at[pl.ds(base, _KS)], idx_s)
                    pltpu.sync_copy(dst_hbm.at[pl.ds(base, _KS)], idx_d)
                    pltpu.async_copy(l_hbm.at[idx_s], lb, sem).wait()
                    pltpu.async_copy(r_hbm.at[idx_d], rb, sem).wait()

                    def _edge(i, c2):
                        g = (i // 16) * 16
                        lane = i - g
                        dv = idx_d[pl.ds(g, 16)]
                        eid = base + g + it
                        okvf = jnp.where(
                            (dv >= r0) & (dv < r0 + _RT) & (eid < e_real),
                            1.0, 0.0)
                        mvec = _bcast_lane(okvf, lane)
                        rowi = _bcast_lane(
                            jnp.clip(dv - r0, 0, _RT - 1), lane)
                        exs = []
                        for h in range(_H):
                            acc = None
                            lvs = []
                            for qq in range(4):
                                q = h * 4 + qq
                                lv = lb[i, 16 * q:16 * (q + 1)]
                                rv = rb[i, 16 * q:16 * (q + 1)]
                                tv = lv + rv
                                tv = jnp.where(tv > 0, tv, 0.2 * tv)
                                tv = tv * attb[16 * q:16 * (q + 1)]
                                acc = tv if acc is None else acc + tv
                                lvs.append(lv)
                            exv = jnp.exp(_allsum(acc)) * mvec
                            for qq in range(4):
                                q = h * 4 + qq
                                plsc.addupdate_scatter(
                                    table, [rowi, 16 * q + it],
                                    exv * lvs[qq])
                            exs.append(exv)
                        exl = None
                        for h in range(_H):
                            ohf = jnp.where(it == h, 1.0, 0.0)
                            cc = exs[h] * ohf
                            exl = cc if exl is None else exl + cc
                        plsc.addupdate_scatter(table, [rowi, _HC + it], exl)
                        return c2

                    lax.fori_loop(0, _KS, _edge, 0)
                    return c

                lax.fori_loop(0, nch, _chunk, 0)
                pltpu.sync_copy(table, out_hbm.at[pl.ds(r0, _RT)])

    return k


@functools.cache
def _build_pool_kernel(w):
    wo = w + 16
    mesh = plsc.VectorSubcoreMesh(core_axis_name="c", subcore_axis_name="s")

    @functools.partial(
        pl.kernel,
        out_type=jax.ShapeDtypeStruct((4 * _G, wo), jnp.float32),
        mesh=mesh,
        compiler_params=pltpu.CompilerParams(needs_layout_passes=False),
        scratch_types=[
            pltpu.VMEM((_KP, w), jnp.float32),
            pltpu.VMEM((_KP,), jnp.int32),
            pltpu.VMEM((w,), jnp.float32),
            pltpu.VMEM((32,), jnp.int32),
            pltpu.VMEM((32,), jnp.int32),
            pltpu.VMEM((_RTP, wo), jnp.float32),
        ],
    )
    def k(x_hbm, seg_hbm, gate_hbm, lo_hbm, hi_hbm, out_hbm,
          xb, segb, gateb, lob, hib, table):
        wid = lax.axis_index("c") * 16 + lax.axis_index("s")
        pltpu.sync_copy(gate_hbm, gateb)
        pltpu.sync_copy(lo_hbm, lob)
        pltpu.sync_copy(hi_hbm, hib)
        it = _iota16()
        part = wid // 8
        rr = wid - part * 8
        r0 = rr * _RTP
        gg = (wid // 16) * 16
        ll = wid - gg
        lo = pl.multiple_of(_extract(lob[pl.ds(gg, 16)], ll), 8)
        hi = _extract(hib[pl.ds(gg, 16)], ll)

        def _zero(i, c):
            for q in range(wo // 16):
                table[i, 16 * q:16 * (q + 1)] = jnp.zeros((16,), jnp.float32)
            return c

        lax.fori_loop(0, _RTP, _zero, 0)
        nch = (hi - lo + _KP - 1) // _KP

        def _chunk(j, c):
            base = pl.multiple_of(lo + j * _KP, 8)
            pltpu.sync_copy(x_hbm.at[pl.ds(base, _KP)], xb)
            pltpu.sync_copy(seg_hbm.at[pl.ds(base, _KP)], segb)

            def _row(i, c2):
                g = (i // 16) * 16
                lane = i - g
                sv = segb[pl.ds(g, 16)]
                eidv = base + g + it
                okvf = jnp.where(
                    (sv >= r0) & (sv < r0 + _RTP) & (eidv < hi), 1.0, 0.0)
                mvec = _bcast_lane(okvf, lane)
                rowi = _bcast_lane(jnp.clip(sv - r0, 0, _RTP - 1), lane)
                acc = None
                xvs = []
                for q in range(w // 16):
                    xv = xb[i, 16 * q:16 * (q + 1)]
                    tv = xv * gateb[16 * q:16 * (q + 1)]
                    acc = tv if acc is None else acc + tv
                    xvs.append(xv)
                exv = jnp.exp(_allsum(acc)) * mvec
                for q in range(w // 16):
                    plsc.addupdate_scatter(
                        table, [rowi, 16 * q + it], exv * xvs[q])
                ohf = jnp.where(it == 0, 1.0, 0.0)
                plsc.addupdate_scatter(table, [rowi, w + it], exv * ohf)
                return c2

            lax.fori_loop(0, _KP, _row, 0)
            return c

        lax.fori_loop(0, nch, _chunk, 0)
        pltpu.sync_copy(table, out_hbm.at[pl.ds(part * _G + r0, _RTP)])

    return k


def _pad_rows(x, n_pad, value=0):
    n = x.shape[0]
    if n == n_pad:
        return x
    pad = [(0, n_pad - n)] + [(0, 0)] * (x.ndim - 1)
    return jnp.pad(x, pad, constant_values=value)


def _edge_plan(src, dst, n_tgt):
    e = src.shape[0]
    e_pad = _rup(e + 2 * _KS, 8)
    srcp = _pad_rows(src.astype(jnp.int32), e_pad)
    dstp = _pad_rows(dst.astype(jnp.int32), e_pad, value=n_tgt)
    order = jnp.argsort(dstp)
    src_s = jnp.take(srcp, order)
    dst_s = jnp.take(dstp, order)
    nr = -(-n_tgt // _RT)
    cuts = jnp.arange(nr + 1, dtype=jnp.int32) * _RT
    b = jnp.searchsorted(dst_s, cuts, side='left').astype(jnp.int32)
    lo = (b[:-1] // 8) * 8
    hi = jnp.minimum(b[1:], e_pad - _KS)
    dst_s = jnp.minimum(dst_s, n_tgt - 1)
    return src_s, dst_s, _pad_rows(lo, 96), _pad_rows(hi, 96), e_pad, e


def _pool_plan(seg, n_rows):
    n_pad = _rup(n_rows + _KP, 8)
    segp = _pad_rows(seg.astype(jnp.int32), n_pad, value=_G)
    cuts = jnp.arange(9, dtype=jnp.int32) * _RTP
    rb = jnp.searchsorted(segp, cuts, side='left').astype(jnp.int32)
    lo_r = rb[:-1]
    hi_r = rb[1:]
    span = hi_r - lo_r
    pts = []
    for p in range(5):
        if p == 0:
            pts.append((lo_r // 8) * 8)
        elif p == 4:
            pts.append(hi_r)
        else:
            pts.append(((lo_r + (span * p) // 4) // 8) * 8)
    lo = jnp.concatenate([pts[p] for p in range(4)])
    hi = jnp.concatenate([pts[p + 1] for p in range(4)])
    return segp, lo.astype(jnp.int32), hi.astype(jnp.int32), n_pad


def _run_layer(xs_in, convp, lnp, proj, plans, first):
    x_pn = xs_in['Package_Name']
    wl_cat = jnp.concatenate([convp[rel]['Wl'] for _, rel, _ in _ETYPES],
                             axis=1)
    l_cat = _mm(x_pn, wl_cat)
    h = {}
    for i, (_, rel, t) in enumerate(_ETYPES):
        n_tgt = xs_in[t].shape[0]
        l = l_cat[:, i * _HC:(i + 1) * _HC]
        r = _mm(xs_in[t], convp[rel]['Wr'])
        src_s, dst_s, lo, hi, e_pad, e_real = plans[rel]
        ek = _build_edge_kernel(n_tgt, e_pad, e_real)
        table = ek(l, r, src_s, dst_s, convp[rel]['att'].reshape(_HC),
                   lo, hi)
        h[t] = _gat_finalize(table, convp[rel]['b'], lnp[t]['g'],
                             lnp[t]['b'], n_tgt)
    if first:
        x2 = _mm(x_pn, proj['W'], proj['b'])
    else:
        x2 = x_pn
    h['Package_Name'] = _ln_relu(x2, lnp['Package_Name']['g'],
                                 lnp['Package_Name']['b'])
    return h


def kernel(xs, edge_attrs, params, edges, node_batch, edge_batch):
    plans = {}
    for _, rel, t in _ETYPES:
        src, dst = edges[rel]
        plans[rel] = _edge_plan(src, dst, xs[t].shape[0])

    h = _run_layer(xs, params['conv1'], params['ln1'], params['proj'],
                   plans, True)
    h = _run_layer(h, params['conv2'], params['ln2'], params['proj'],
                   plans, False)

    ntabs = []
    for nt, gate in params['node_gate'].items():
        x = h[nt]
        segp, lo, hi, n_pad = _pool_plan(node_batch[nt], x.shape[0])
        pk = _build_pool_kernel(_HC)
        tab = pk(_pad_rows(x, n_pad), segp, gate, lo, hi)
        ntabs.append(tab.reshape(4, _G, _TW))
    etabs = []
    for rel, gate in params['edge_gate'].items():
        ea = edge_attrs[rel]
        segp, lo, hi, n_pad = _pool_plan(edge_batch[rel], ea.shape[0])
        pk = _build_pool_kernel(16)
        tab = pk(_pad_rows(ea, n_pad), segp, gate, lo, hi)
        etabs.append(tab.reshape(4, _G, 32))

    return _readout(ntabs, etabs, params['cls']['W'], params['cls']['b'])
